# baseline (speedup 1.0000x reference)
import math

import jax
import jax.numpy as jnp
from jax.experimental import pallas as pl
from jax.experimental.pallas import tpu as pltpu

_CH = 256
_NBUF = 3


def kernel(input, pe):
    batch, seq, d_model = input.shape
    scale = math.sqrt(pe.shape[1])
    nch = seq // _CH

    def body(x_hbm, pe_hbm, o_hbm, xb, pb, ob, xsem, psem, osem):
        i = pl.program_id(0)

        def start_in(c, slot):
            pltpu.make_async_copy(
                x_hbm.at[:, pl.ds(c * _CH, _CH), :], xb.at[slot], xsem.at[slot]
            ).start()
            pltpu.make_async_copy(
                pe_hbm.at[pl.ds(c * _CH, _CH), :], pb.at[slot], psem.at[slot]
            ).start()

        @pl.when(i == 0)
        def _():
            for k in range(_NBUF):
                start_in(k, k)

        slot = jax.lax.rem(i, _NBUF)
        pltpu.make_async_copy(
            x_hbm.at[:, pl.ds(i * _CH, _CH), :], xb.at[slot], xsem.at[slot]
        ).wait()
        pltpu.make_async_copy(
            pe_hbm.at[pl.ds(i * _CH, _CH), :], pb.at[slot], psem.at[slot]
        ).wait()

        @pl.when(i >= _NBUF)
        def _():
            pltpu.make_async_copy(
                ob.at[slot], o_hbm.at[:, pl.ds((i - _NBUF) * _CH, _CH), :], osem.at[slot]
            ).wait()

        ob[slot] = xb[slot] * scale + pb[slot][None, :, :]

        pltpu.make_async_copy(
            ob.at[slot], o_hbm.at[:, pl.ds(i * _CH, _CH), :], osem.at[slot]
        ).start()

        @pl.when(i + _NBUF < nch)
        def _():
            start_in(i + _NBUF, slot)

        @pl.when(i == nch - 1)
        def _():
            for k in range(_NBUF):
                pltpu.make_async_copy(
                    ob.at[k], o_hbm.at[:, pl.ds(k * _CH, _CH), :], osem.at[k]
                ).wait()

    return pl.pallas_call(
        body,
        grid=(nch,),
        in_specs=[
            pl.BlockSpec(memory_space=pl.ANY),
            pl.BlockSpec(memory_space=pl.ANY),
        ],
        out_specs=pl.BlockSpec(memory_space=pl.ANY),
        out_shape=jax.ShapeDtypeStruct((batch, seq, d_model), input.dtype),
        scratch_shapes=[
            pltpu.VMEM((_NBUF, batch, _CH, d_model), input.dtype),
            pltpu.VMEM((_NBUF, _CH, d_model), pe.dtype),
            pltpu.VMEM((_NBUF, batch, _CH, d_model), input.dtype),
            pltpu.SemaphoreType.DMA((_NBUF,)),
            pltpu.SemaphoreType.DMA((_NBUF,)),
            pltpu.SemaphoreType.DMA((_NBUF,)),
        ],
    )(input, pe)


# speedup vs baseline: 1.0012x; 1.0012x over previous
import math

import jax
from jax.experimental import pallas as pl


def _pe_add_kernel(x_ref, pe_ref, o_ref, *, scale):
    o_ref[...] = x_ref[...] * scale + pe_ref[...][None, :, :]


def kernel(input, pe):
    batch, seq, d_model = input.shape
    scale = math.sqrt(pe.shape[1])
    blk = 512
    grid = (seq // blk,)
    return pl.pallas_call(
        lambda x_ref, pe_ref, o_ref: _pe_add_kernel(x_ref, pe_ref, o_ref, scale=scale),
        grid=grid,
        in_specs=[
            pl.BlockSpec((batch, blk, d_model), lambda i: (0, i, 0)),
            pl.BlockSpec((blk, d_model), lambda i: (i, 0)),
        ],
        out_specs=pl.BlockSpec((batch, blk, d_model), lambda i: (0, i, 0)),
        out_shape=jax.ShapeDtypeStruct((batch, seq, d_model), input.dtype),
    )(input, pe)
